# baseline (speedup 1.0000x reference)
import functools

import jax
import jax.numpy as jnp
from jax import lax
from jax.experimental import pallas as pl
from jax.experimental.pallas import tpu as pltpu
from jax.experimental.pallas import tpu_sc as plsc

N = 100000
C = 9
V = 200
D = 512
W = D // 2
B = 16
G = B * C
H = G // 2
NBLK = N // B
NW = 32
L = 16

@functools.partial(
    pl.kernel,
    out_type=jax.ShapeDtypeStruct((N, D), jnp.float32),
    mesh=plsc.VectorSubcoreMesh(core_axis_name="c", subcore_axis_name="s"),
    scratch_types=[
        pltpu.VMEM((2 * G,), jnp.int32),
        pltpu.VMEM((2, G, W), jnp.int32),
        pltpu.VMEM((2, B, D), jnp.float32),
        pltpu.SemaphoreType.DMA((2,)),
        pltpu.SemaphoreType.DMA((2,)),
        pltpu.SemaphoreType.DMA((2,)),
    ],
)
def _atom_encoder_sc(idx_hbm, tabs_hbm, out_hbm, idx_v, rows_v, out_v,
                     sem_idx, sem_g, sem_o):
    w = lax.axis_index("s") * 2 + lax.axis_index("c")
    nblk_w = jnp.where(w < 10, 196, 195)
    blk0 = w * 195 + jnp.minimum(w, 10)

    def idx_copy(blk, slot, sem):
        return pltpu.make_async_copy(
            idx_hbm.at[pl.ds(blk * G, G)],
            idx_v.at[pl.ds(pl.multiple_of(slot * G, 8), G)], sem)

    def gathers(slot, sem):
        return [
            pltpu.make_async_copy(
                tabs_hbm.at[idx_v.at[
                    pl.ds(pl.multiple_of(slot * G + h * H, 8), H)]],
                rows_v.at[slot, pl.ds(h * H, H)], sem)
            for h in range(2)
        ]

    def out_store(slot, blk, sem):
        return pltpu.make_async_copy(
            out_v.at[slot], out_hbm.at[pl.ds(blk * B, B)], sem)

    idx_copy(blk0, 0, sem_idx.at[0]).start()
    idx_copy(blk0, 0, sem_idx.at[0]).wait()
    for g in gathers(0, sem_g.at[0]):
        g.start()

    @pl.when(nblk_w > 1)
    def _():
        idx_copy(blk0 + 1, 1, sem_idx.at[1]).start()

    def block_step(k, carry):
        blk = blk0 + k
        buf = lax.rem(k, 2)
        nbuf = 1 - buf

        for g in gathers(buf, sem_g.at[buf]):
            g.wait()

        @pl.when(k + 2 < nblk_w)
        def _():
            idx_copy(blk + 2, buf, sem_idx.at[buf]).start()

        @pl.when(k + 1 < nblk_w)
        def _():
            idx_copy(blk + 1, nbuf, sem_idx.at[nbuf]).wait()
            for g in gathers(nbuf, sem_g.at[nbuf]):
                g.start()

        @pl.when(k >= 2)
        def _():
            out_store(buf, blk - 2, sem_o.at[buf]).wait()

        def row_step(n, c1):
            def col_step(c, c2):
                for u in range(2):
                    cc = c * 2 + u
                    sl = pl.ds(cc * L, L)
                    wd = rows_v[buf, n * C, sl]
                    lo = lax.bitcast_convert_type(wd << 16, jnp.float32)
                    hi = lax.bitcast_convert_type(wd, jnp.float32)
                    for i in range(1, C):
                        wd = rows_v[buf, n * C + i, sl]
                        lo = lo + lax.bitcast_convert_type(
                            wd << 16, jnp.float32)
                        hi = hi + lax.bitcast_convert_type(wd, jnp.float32)
                    out_v[buf, n, sl] = lo
                    out_v[buf, n, pl.ds(W + cc * L, L)] = hi
                return c2

            return lax.fori_loop(0, W // L // 2, col_step, c1)

        lax.fori_loop(0, B, row_step, 0)
        out_store(buf, blk, sem_o.at[buf]).start()
        return carry

    lax.fori_loop(0, nblk_w, block_step, 0)

    p = lax.rem(nblk_w, 2)
    out_store(p, blk0 + nblk_w - 2, sem_o.at[p]).wait()
    out_store(1 - p, blk0 + nblk_w - 1, sem_o.at[1 - p]).wait()


def kernel(x, tables):
    offs = (jnp.arange(C, dtype=jnp.int32) * V)[None, :]
    idx = (x.astype(jnp.int32) + offs).reshape(N * C)
    blk_of_row = jnp.arange(N, dtype=jnp.int32) // B
    wid = jnp.where(blk_of_row < 10 * 196, blk_of_row // 196,
                    10 + (blk_of_row - 10 * 196) // 195)
    idx = idx + jnp.repeat(wid * (C * V), C, total_repeat_length=N * C)
    tabs16 = tables.astype(jnp.bfloat16).reshape(C * V, D)
    pairs = jnp.stack([tabs16[:, :W], tabs16[:, W:]], axis=-1)
    tabs = lax.bitcast_convert_type(pairs, jnp.int32)
    tabs_rep = jnp.tile(tabs, (NW, 1))
    return _atom_encoder_sc(idx, tabs_rep)


# speedup vs baseline: 11.7714x; 11.7714x over previous
import functools

import jax
import jax.numpy as jnp
from jax import lax
from jax.experimental import pallas as pl
from jax.experimental.pallas import tpu as pltpu
from jax.experimental.pallas import tpu_sc as plsc

N = 100000
C = 9
V = 200
D = 512
W = D // 2
B = 16
G = B * C
H = G // 2
NBLK = N // B
NW = 32
L = 16

@functools.partial(
    pl.kernel,
    out_type=jax.ShapeDtypeStruct((N, D), jnp.float32),
    mesh=plsc.VectorSubcoreMesh(core_axis_name="c", subcore_axis_name="s"),
    scratch_types=[
        pltpu.VMEM((2 * G,), jnp.int32),
        pltpu.VMEM((2, G, W), jnp.int32),
        pltpu.VMEM((2, B, D), jnp.float32),
        pltpu.SemaphoreType.DMA((2,)),
        pltpu.SemaphoreType.DMA((2,)),
        pltpu.SemaphoreType.DMA((2,)),
    ],
)
def _atom_encoder_sc(idx_hbm, tabs_hbm, out_hbm, idx_v, rows_v, out_v,
                     sem_idx, sem_g, sem_o):
    w = lax.axis_index("s") * 2 + lax.axis_index("c")
    nblk_w = jnp.where(w < 10, 196, 195)
    blk0 = w * 195 + jnp.minimum(w, 10)

    def idx_copy(blk, slot, sem):
        return pltpu.make_async_copy(
            idx_hbm.at[pl.ds(blk * G, G)],
            idx_v.at[pl.ds(pl.multiple_of(slot * G, 8), G)], sem)

    def gathers(slot, sem):
        return [
            pltpu.make_async_copy(
                tabs_hbm.at[idx_v.at[
                    pl.ds(pl.multiple_of(slot * G + h * H, 8), H)]],
                rows_v.at[slot, pl.ds(h * H, H)], sem)
            for h in range(2)
        ]

    def out_store(slot, blk, sem):
        return pltpu.make_async_copy(
            out_v.at[slot], out_hbm.at[pl.ds(blk * B, B)], sem)

    idx_copy(blk0, 0, sem_idx.at[0]).start()
    idx_copy(blk0, 0, sem_idx.at[0]).wait()
    for g in gathers(0, sem_g.at[0]):
        g.start()

    @pl.when(nblk_w > 1)
    def _():
        idx_copy(blk0 + 1, 1, sem_idx.at[1]).start()

    def block_step(k, carry):
        blk = blk0 + k
        buf = lax.rem(k, 2)
        nbuf = 1 - buf

        for g in gathers(buf, sem_g.at[buf]):
            g.wait()

        @pl.when(k + 2 < nblk_w)
        def _():
            idx_copy(blk + 2, buf, sem_idx.at[buf]).start()

        @pl.when(k + 1 < nblk_w)
        def _():
            idx_copy(blk + 1, nbuf, sem_idx.at[nbuf]).wait()
            for g in gathers(nbuf, sem_g.at[nbuf]):
                g.start()

        @pl.when(k >= 2)
        def _():
            out_store(buf, blk - 2, sem_o.at[buf]).wait()

        def tree_sum(vals):
            while len(vals) > 1:
                nxt = [a + b for a, b in zip(vals[::2], vals[1::2])]
                if len(vals) % 2:
                    nxt.append(vals[-1])
                vals = nxt
            return vals[0]

        def row_step(n, c1):
            def col_step(c, c2):
                for u in range(2):
                    cc = c * 2 + u
                    sl = pl.ds(cc * L, L)
                    wds = [rows_v[buf, n * C + i, sl] for i in range(C)]
                    lo = tree_sum([
                        lax.bitcast_convert_type(wd << 16, jnp.float32)
                        for wd in wds])
                    hi = tree_sum([
                        lax.bitcast_convert_type(wd, jnp.float32)
                        for wd in wds])
                    out_v[buf, n, sl] = lo
                    out_v[buf, n, pl.ds(W + cc * L, L)] = hi
                return c2

            return lax.fori_loop(0, W // L // 2, col_step, c1)

        lax.fori_loop(0, B, row_step, 0)
        out_store(buf, blk, sem_o.at[buf]).start()
        return carry

    lax.fori_loop(0, nblk_w, block_step, 0)

    p = lax.rem(nblk_w, 2)
    out_store(p, blk0 + nblk_w - 2, sem_o.at[p]).wait()
    out_store(1 - p, blk0 + nblk_w - 1, sem_o.at[1 - p]).wait()


def kernel(x, tables):
    offs = (jnp.arange(C, dtype=jnp.int32) * V)[None, :]
    idx = (x.astype(jnp.int32) + offs).reshape(N * C)
    tabs16 = tables.astype(jnp.bfloat16).reshape(C * V, D)
    pairs = jnp.stack([tabs16[:, :W], tabs16[:, W:]], axis=-1)
    tabs = lax.bitcast_convert_type(pairs, jnp.int32)
    return _atom_encoder_sc(idx, tabs)


# speedup vs baseline: 12.1920x; 1.0357x over previous
import functools

import jax
import jax.numpy as jnp
from jax import lax
from jax.experimental import pallas as pl
from jax.experimental.pallas import tpu as pltpu
from jax.experimental.pallas import tpu_sc as plsc

N = 100000
C = 9
V = 200
D = 512
W = D // 2
B = 16
G = B * C
H = G // 2
NBLK = N // B
NW = 32
L = 16

@functools.partial(
    pl.kernel,
    out_type=jax.ShapeDtypeStruct((N, D), jnp.float32),
    mesh=plsc.VectorSubcoreMesh(core_axis_name="c", subcore_axis_name="s"),
    scratch_types=[
        pltpu.VMEM((2 * G,), jnp.int32),
        pltpu.VMEM((2, G, W), jnp.int32),
        pltpu.VMEM((2, B, D), jnp.float32),
        pltpu.SemaphoreType.DMA((2,)),
        pltpu.SemaphoreType.DMA((2,)),
        pltpu.SemaphoreType.DMA((2,)),
    ],
)
def _atom_encoder_sc(idx_hbm, tabs_hbm, out_hbm, idx_v, rows_v, out_v,
                     sem_idx, sem_g, sem_o):
    w = lax.axis_index("s") * 2 + lax.axis_index("c")
    nblk_w = jnp.where(w < 10, 196, 195)
    blk0 = w * 195 + jnp.minimum(w, 10)

    def idx_copy(blk, slot, sem):
        return pltpu.make_async_copy(
            idx_hbm.at[pl.ds(blk * G, G)],
            idx_v.at[pl.ds(pl.multiple_of(slot * G, 8), G)], sem)

    def gathers(slot, sem):
        return [
            pltpu.make_async_copy(
                tabs_hbm.at[idx_v.at[
                    pl.ds(pl.multiple_of(slot * G + h * H, 8), H)]],
                rows_v.at[slot, pl.ds(h * H, H)], sem)
            for h in range(2)
        ]

    def out_store(slot, blk, sem):
        return pltpu.make_async_copy(
            out_v.at[slot], out_hbm.at[pl.ds(blk * B, B)], sem)

    idx_copy(blk0, 0, sem_idx.at[0]).start()
    idx_copy(blk0, 0, sem_idx.at[0]).wait()
    for g in gathers(0, sem_g.at[0]):
        g.start()

    @pl.when(nblk_w > 1)
    def _():
        idx_copy(blk0 + 1, 1, sem_idx.at[1]).start()

    def block_step(k, carry):
        blk = blk0 + k
        buf = lax.rem(k, 2)
        nbuf = 1 - buf

        for g in gathers(buf, sem_g.at[buf]):
            g.wait()

        @pl.when(k + 2 < nblk_w)
        def _():
            idx_copy(blk + 2, buf, sem_idx.at[buf]).start()

        @pl.when(k + 1 < nblk_w)
        def _():
            idx_copy(blk + 1, nbuf, sem_idx.at[nbuf]).wait()
            for g in gathers(nbuf, sem_g.at[nbuf]):
                g.start()

        @pl.when(k >= 2)
        def _():
            out_store(buf, blk - 2, sem_o.at[buf]).wait()

        def tree_sum(vals):
            while len(vals) > 1:
                nxt = [a + b for a, b in zip(vals[::2], vals[1::2])]
                if len(vals) % 2:
                    nxt.append(vals[-1])
                vals = nxt
            return vals[0]

        def row_step(n, c1):
            for cc in range(W // L):
                sl = pl.ds(cc * L, L)
                wds = [rows_v[buf, n * C + i, sl] for i in range(C)]
                lo = tree_sum([
                    lax.bitcast_convert_type(wd << 16, jnp.float32)
                    for wd in wds])
                hi = tree_sum([
                    lax.bitcast_convert_type(wd, jnp.float32)
                    for wd in wds])
                out_v[buf, n, sl] = lo
                out_v[buf, n, pl.ds(W + cc * L, L)] = hi
            return c1

        lax.fori_loop(0, B, row_step, 0)
        out_store(buf, blk, sem_o.at[buf]).start()
        return carry

    lax.fori_loop(0, nblk_w, block_step, 0)

    p = lax.rem(nblk_w, 2)
    out_store(p, blk0 + nblk_w - 2, sem_o.at[p]).wait()
    out_store(1 - p, blk0 + nblk_w - 1, sem_o.at[1 - p]).wait()


def kernel(x, tables):
    offs = (jnp.arange(C, dtype=jnp.int32) * V)[None, :]
    idx = (x.astype(jnp.int32) + offs).reshape(N * C)
    tabs16 = tables.astype(jnp.bfloat16).reshape(C * V, D)
    pairs = jnp.stack([tabs16[:, :W], tabs16[:, W:]], axis=-1)
    tabs = lax.bitcast_convert_type(pairs, jnp.int32)
    return _atom_encoder_sc(idx, tabs)


# speedup vs baseline: 12.3513x; 1.0131x over previous
import functools

import jax
import jax.numpy as jnp
from jax import lax
from jax.experimental import pallas as pl
from jax.experimental.pallas import tpu as pltpu
from jax.experimental.pallas import tpu_sc as plsc

N = 100000
C = 9
V = 200
D = 512
W = D // 2
B = 16
G = B * C
H = G // 2
NBLK = N // B
NW = 32
L = 16

@functools.partial(
    pl.kernel,
    out_type=jax.ShapeDtypeStruct((N, D), jnp.float32),
    mesh=plsc.VectorSubcoreMesh(core_axis_name="c", subcore_axis_name="s"),
    scratch_types=[
        pltpu.VMEM((2 * G,), jnp.int32),
        pltpu.VMEM((2, G, W), jnp.int32),
        pltpu.VMEM((2, B, D), jnp.float32),
        pltpu.SemaphoreType.DMA((2,)),
        pltpu.SemaphoreType.DMA((2,)),
        pltpu.SemaphoreType.DMA((2,)),
    ],
)
def _atom_encoder_sc(idx_hbm, tabs_hbm, out_hbm, idx_v, rows_v, out_v,
                     sem_idx, sem_g, sem_o):
    w = lax.axis_index("s") * 2 + lax.axis_index("c")
    nblk_w = jnp.where(w < 10, 196, 195)
    blk0 = w * 195 + jnp.minimum(w, 10)

    def idx_copy(blk, slot, sem):
        return pltpu.make_async_copy(
            idx_hbm.at[pl.ds(blk * G, G)],
            idx_v.at[pl.ds(pl.multiple_of(slot * G, 8), G)], sem)

    def gathers(slot, sem):
        return [
            pltpu.make_async_copy(
                tabs_hbm.at[idx_v.at[
                    pl.ds(pl.multiple_of(slot * G + h * H, 8), H)]],
                rows_v.at[slot, pl.ds(h * H, H)], sem)
            for h in range(2)
        ]

    def out_store(slot, blk, sem):
        return pltpu.make_async_copy(
            out_v.at[slot], out_hbm.at[pl.ds(blk * B, B)], sem)

    idx_copy(blk0, 0, sem_idx.at[0]).start()
    idx_copy(blk0, 0, sem_idx.at[0]).wait()
    for g in gathers(0, sem_g.at[0]):
        g.start()

    @pl.when(nblk_w > 1)
    def _():
        idx_copy(blk0 + 1, 1, sem_idx.at[1]).start()

    def block_step(k, carry):
        blk = blk0 + k
        buf = lax.rem(k, 2)
        nbuf = 1 - buf

        for g in gathers(buf, sem_g.at[buf]):
            g.wait()

        @pl.when(k + 2 < nblk_w)
        def _():
            idx_copy(blk + 2, buf, sem_idx.at[buf]).start()

        @pl.when(k + 1 < nblk_w)
        def _():
            idx_copy(blk + 1, nbuf, sem_idx.at[nbuf]).wait()
            for g in gathers(nbuf, sem_g.at[nbuf]):
                g.start()

        @pl.when(k >= 2)
        def _():
            out_store(buf, blk - 2, sem_o.at[buf]).wait()

        def tree_sum(vals):
            while len(vals) > 1:
                nxt = [a + b for a, b in zip(vals[::2], vals[1::2])]
                if len(vals) % 2:
                    nxt.append(vals[-1])
                vals = nxt
            return vals[0]

        def row_step(m, c1):
            for j in range(2):
                n = m * 2 + j
                for cc in range(W // L):
                    sl = pl.ds(cc * L, L)
                    wds = [rows_v[buf, n * C + i, sl] for i in range(C)]
                    lo = tree_sum([
                        lax.bitcast_convert_type(wd << 16, jnp.float32)
                        for wd in wds])
                    hi = tree_sum([
                        lax.bitcast_convert_type(wd, jnp.float32)
                        for wd in wds])
                    out_v[buf, n, sl] = lo
                    out_v[buf, n, pl.ds(W + cc * L, L)] = hi
            return c1

        lax.fori_loop(0, B // 2, row_step, 0)
        out_store(buf, blk, sem_o.at[buf]).start()
        return carry

    lax.fori_loop(0, nblk_w, block_step, 0)

    p = lax.rem(nblk_w, 2)
    out_store(p, blk0 + nblk_w - 2, sem_o.at[p]).wait()
    out_store(1 - p, blk0 + nblk_w - 1, sem_o.at[1 - p]).wait()


def kernel(x, tables):
    offs = (jnp.arange(C, dtype=jnp.int32) * V)[None, :]
    idx = (x.astype(jnp.int32) + offs).reshape(N * C)
    tabs16 = tables.astype(jnp.bfloat16).reshape(C * V, D)
    pairs = jnp.stack([tabs16[:, :W], tabs16[:, W:]], axis=-1)
    tabs = lax.bitcast_convert_type(pairs, jnp.int32)
    return _atom_encoder_sc(idx, tabs)
